# baseline (speedup 1.0000x reference)
import functools

import jax
import jax.numpy as jnp
from jax import lax
from jax.experimental import pallas as pl
from jax.experimental.pallas import tpu as pltpu
from jax.experimental.pallas import tpu_sc as plsc

SEQ = 200
BATCH = 4096
EMB = 64
HID = 20
NUM_LABEL = 10

_NUM_CORES = 2
_NUM_SUBCORES = 16
_NUM_WORKERS = _NUM_CORES * _NUM_SUBCORES
_CHUNK = BATCH // _NUM_WORKERS
_LANES = 16


def _gather_pool_body(docs_hbm, table_hbm, out_hbm, docs_v, rows0, rows1,
                      acc, sem0, sem1):
    wid = lax.axis_index("s") * _NUM_CORES + lax.axis_index("c")
    base = wid * _CHUNK

    pltpu.sync_copy(docs_hbm.at[:, pl.ds(base, _CHUNK)], docs_v)

    zero = jnp.zeros((_LANES,), jnp.float32)

    def zrow(r, carry):
        for k in range(EMB // _LANES):
            acc[r, pl.ds(k * _LANES, _LANES)] = zero
        return carry

    lax.fori_loop(0, _CHUNK, zrow, 0, unroll=4)

    pltpu.async_copy(table_hbm.at[docs_v.at[0]], rows0, sem0)
    pltpu.async_copy(table_hbm.at[docs_v.at[1]], rows1, sem1)

    def accumulate(rows):
        def row(r, carry):
            for k in range(EMB // _LANES):
                sl = pl.ds(k * _LANES, _LANES)
                plsc.addupdate(acc.at[r, sl], rows[r, sl])
            return carry

        lax.fori_loop(0, _CHUNK, row, 0, unroll=4)

    def pair(p, carry):
        s0 = 2 * p
        for buf, sem, s in ((rows0, sem0, s0), (rows1, sem1, s0 + 1)):
            pltpu.make_async_copy(table_hbm.at[docs_v.at[s]], buf, sem).wait()
            accumulate(buf)

            @pl.when(s + 2 < SEQ)
            def _():
                pltpu.async_copy(table_hbm.at[docs_v.at[s + 2]], buf, sem)

        return carry

    lax.fori_loop(0, SEQ // 2, pair, 0)

    pltpu.sync_copy(acc, out_hbm.at[pl.ds(base, _CHUNK), :])


@jax.jit
def _gather_pool(docs, emb_table):
    mesh = plsc.VectorSubcoreMesh(core_axis_name="c", subcore_axis_name="s")
    kern = functools.partial(
        pl.kernel,
        mesh=mesh,
        out_type=jax.ShapeDtypeStruct((BATCH, EMB), jnp.float32),
        scratch_types=[
            pltpu.VMEM((SEQ, _CHUNK), jnp.int32),
            pltpu.VMEM((_CHUNK, EMB), jnp.float32),
            pltpu.VMEM((_CHUNK, EMB), jnp.float32),
            pltpu.VMEM((_CHUNK, EMB), jnp.float32),
            pltpu.SemaphoreType.DMA,
            pltpu.SemaphoreType.DMA,
        ],
        compiler_params=pltpu.CompilerParams(use_tc_tiling_on_sc=False),
    )(_gather_pool_body)
    return kern(docs, emb_table)


def _mlp_body(x_ref, w1t_ref, b1_ref, w2t_ref, b2_ref, o_ref):
    x = x_ref[...] * (1.0 / SEQ)
    h = jnp.dot(x, w1t_ref[...], preferred_element_type=jnp.float32)
    h = jnp.maximum(h + b1_ref[...], 0.0)
    o = jnp.dot(h, w2t_ref[...], preferred_element_type=jnp.float32)
    o_ref[...] = o + b2_ref[...]


@jax.jit
def _mlp(pooled_sum, W1, b1, W2, b2):
    return pl.pallas_call(
        _mlp_body,
        out_shape=jax.ShapeDtypeStruct((BATCH, NUM_LABEL), jnp.float32),
    )(pooled_sum, W1.T, b1[None, :], W2.T, b2[None, :])


def kernel(docs, emb_table, W1, b1, W2, b2):
    pooled_sum = _gather_pool(docs, emb_table)
    return _mlp(pooled_sum, W1, b1, W2, b2)
